# Initial kernel scaffold; baseline (speedup 1.0000x reference)
import jax
import jax.numpy as jnp
from jax.experimental import pallas as pl


def kernel(edge_index, mu, log_sigma):
    raise NotImplementedError("write your pallas kernel here")


# speedup vs baseline: 2.5365x; 2.5365x over previous
import functools

import jax
import jax.numpy as jnp
from jax import lax
from jax.experimental import pallas as pl
from jax.experimental.pallas import tpu as pltpu
from jax.experimental.pallas import tpu_sc as plsc

_N = 10000
_D = 128
_E = 320000
_NC = 2
_NS = 16
_NW = _NC * _NS
_EPW = _E // _NW
_B = 80
_NCHUNK = _EPW // _B
_LANES = 16


def _sc_dist_body(src_hbm, dst_hbm, mu_hbm, out_hbm,
                  sidx, didx, srows, drows, part, dist_v, sem_s, sem_d):
    wid = lax.axis_index("s") * _NC + lax.axis_index("c")
    base0 = wid * _EPW
    pltpu.sync_copy(src_hbm.at[pl.ds(base0, _EPW)], sidx)
    pltpu.sync_copy(dst_hbm.at[pl.ds(base0, _EPW)], didx)
    lane = jnp.arange(_LANES, dtype=jnp.int32)

    def chunk_body(i, carry):
        off = i * _B
        cp_s = pltpu.async_copy(mu_hbm.at[sidx.at[pl.ds(off, _B)]], srows, sem_s)
        cp_d = pltpu.async_copy(mu_hbm.at[didx.at[pl.ds(off, _B)]], drows, sem_d)
        cp_s.wait()
        cp_d.wait()
        for e in range(_B):
            acc = None
            for j in range(_D // _LANES):
                s = srows[e, pl.ds(j * _LANES, _LANES)]
                t = drows[e, pl.ds(j * _LANES, _LANES)]
                df = s - t
                acc = df * df if acc is None else acc + df * df
            part[pl.ds(e * _LANES, _LANES)] = acc
        lane16 = lane * _LANES
        for g in range(_B // _LANES):
            acc = jnp.zeros((_LANES,), jnp.float32)
            for k in range(_LANES):
                idx = lane16 + (g * _LANES * _LANES + k)
                acc = acc + plsc.load_gather(part, [idx])
            dist_v[pl.ds(off + g * _LANES, _LANES)] = acc
        return carry

    lax.fori_loop(0, _NCHUNK, chunk_body, 0)
    pltpu.sync_copy(dist_v, out_hbm.at[pl.ds(base0, _EPW)])


_sc_dist = functools.partial(
    pl.kernel,
    out_type=jax.ShapeDtypeStruct((_E,), jnp.float32),
    mesh=plsc.VectorSubcoreMesh(core_axis_name="c", subcore_axis_name="s"),
    compiler_params=pltpu.CompilerParams(needs_layout_passes=False),
    scratch_types=[
        pltpu.VMEM((_EPW,), jnp.int32),
        pltpu.VMEM((_EPW,), jnp.int32),
        pltpu.VMEM((_B, _D), jnp.float32),
        pltpu.VMEM((_B, _D), jnp.float32),
        pltpu.VMEM((_B * _LANES,), jnp.float32),
        pltpu.VMEM((_EPW,), jnp.float32),
        pltpu.SemaphoreType.DMA,
        pltpu.SemaphoreType.DMA,
    ],
)(_sc_dist_body)


def _tc_epilogue(dist_ref, mu_ref, ls_ref, out_ref):
    d = dist_ref[...]
    p = jax.nn.sigmoid(-d)
    nll = -jnp.sum(jnp.log(p))
    m = mu_ref[...]
    ls = ls_ref[...]
    sig = jnp.exp(ls)
    kl = 0.5 * jnp.sum(sig * sig + m * m - ls - 1.0)
    out_ref[0, 0] = nll + kl


def kernel(edge_index, mu, log_sigma):
    src = edge_index[0]
    dst = edge_index[1]
    dist = _sc_dist(src, dst, mu)
    dist2 = dist.reshape(_E // _D, _D)
    out = pl.pallas_call(
        _tc_epilogue,
        out_specs=pl.BlockSpec(memory_space=pltpu.SMEM),
        out_shape=jax.ShapeDtypeStruct((1, 1), jnp.float32),
    )(dist2, mu, log_sigma)
    return out[0, 0]


# speedup vs baseline: 3.2244x; 1.2712x over previous
import functools

import jax
import jax.numpy as jnp
from jax import lax
from jax.experimental import pallas as pl
from jax.experimental.pallas import tpu as pltpu
from jax.experimental.pallas import tpu_sc as plsc

_N = 10000
_D = 128
_E = 320000
_NC = 2
_NS = 16
_NW = _NC * _NS
_EPW = _E // _NW
_B = 80
_NCHUNK = _EPW // _B
_LANES = 16


def _sc_dist_body(src_hbm, dst_hbm, mu_hbm, out_hbm,
                  sidx, didx, srows, drows, dist_v, sem_s, sem_d):
    wid = lax.axis_index("s") * _NC + lax.axis_index("c")
    base0 = wid * _EPW
    pltpu.sync_copy(src_hbm.at[pl.ds(base0, _EPW)], sidx)
    pltpu.sync_copy(dst_hbm.at[pl.ds(base0, _EPW)], didx)
    lane = jnp.arange(_LANES, dtype=jnp.int32)

    def chunk_body(i, carry):
        off = i * _B
        cp_s = pltpu.async_copy(mu_hbm.at[sidx.at[pl.ds(off, _B)]], srows, sem_s)
        cp_d = pltpu.async_copy(mu_hbm.at[didx.at[pl.ds(off, _B)]], drows, sem_d)
        cp_s.wait()
        cp_d.wait()
        for g in range(_B // _LANES):
            res = jnp.zeros((_LANES,), jnp.float32)
            for l in range(_LANES):
                e = g * _LANES + l
                acc = None
                for j in range(_D // _LANES):
                    s = srows[e, pl.ds(j * _LANES, _LANES)]
                    t = drows[e, pl.ds(j * _LANES, _LANES)]
                    df = s - t
                    acc = df * df if acc is None else acc + df * df
                for sh in (8, 4, 2, 1):
                    acc = acc + _lane_perm(acc, lane ^ sh)
                res = jnp.where(lane == l, acc, res)
            dist_v[pl.ds(off + g * _LANES, _LANES)] = res
        return carry

    lax.fori_loop(0, _NCHUNK, chunk_body, 0)
    pltpu.sync_copy(dist_v, out_hbm.at[pl.ds(base0, _EPW)])


_GDN = lax.GatherDimensionNumbers(
    offset_dims=(), collapsed_slice_dims=(0,), start_index_map=(0,))


def _lane_perm(v, idx):
    return lax.gather(v, idx[:, None], dimension_numbers=_GDN,
                      slice_sizes=(1,), mode=lax.GatherScatterMode.PROMISE_IN_BOUNDS)


_sc_dist = functools.partial(
    pl.kernel,
    out_type=jax.ShapeDtypeStruct((_E,), jnp.float32),
    mesh=plsc.VectorSubcoreMesh(core_axis_name="c", subcore_axis_name="s"),
    compiler_params=pltpu.CompilerParams(needs_layout_passes=False),
    scratch_types=[
        pltpu.VMEM((_EPW,), jnp.int32),
        pltpu.VMEM((_EPW,), jnp.int32),
        pltpu.VMEM((_B, _D), jnp.float32),
        pltpu.VMEM((_B, _D), jnp.float32),
        pltpu.VMEM((_EPW,), jnp.float32),
        pltpu.SemaphoreType.DMA,
        pltpu.SemaphoreType.DMA,
    ],
)(_sc_dist_body)


def _tc_epilogue(dist_ref, mu_ref, ls_ref, out_ref):
    d = dist_ref[...]
    p = jax.nn.sigmoid(-d)
    nll = -jnp.sum(jnp.log(p))
    m = mu_ref[...]
    ls = ls_ref[...]
    sig = jnp.exp(ls)
    kl = 0.5 * jnp.sum(sig * sig + m * m - ls - 1.0)
    out_ref[0, 0] = nll + kl


def kernel(edge_index, mu, log_sigma):
    src = edge_index[0]
    dst = edge_index[1]
    dist = _sc_dist(src, dst, mu)
    dist2 = dist.reshape(_E // _D, _D)
    out = pl.pallas_call(
        _tc_epilogue,
        out_specs=pl.BlockSpec(memory_space=pltpu.SMEM),
        out_shape=jax.ShapeDtypeStruct((1, 1), jnp.float32),
    )(dist2, mu, log_sigma)
    return out[0, 0]


# speedup vs baseline: 4.2193x; 1.3086x over previous
import functools

import jax
import jax.numpy as jnp
from jax import lax
from jax.experimental import pallas as pl
from jax.experimental.pallas import tpu as pltpu
from jax.experimental.pallas import tpu_sc as plsc

_N = 10000
_D = 128
_E = 320000
_NC = 2
_NS = 16
_NW = _NC * _NS
_EPW = _E // _NW
_B = 80
_NCHUNK = _EPW // _B
_LANES = 16


def _sc_dist_body(src_hbm, dst_hbm, mu_hbm, out_hbm,
                  sidx, didx, srows0, drows0, srows1, drows1, dist_v,
                  sem_s0, sem_d0, sem_s1, sem_d1):
    wid = lax.axis_index("s") * _NC + lax.axis_index("c")
    base0 = wid * _EPW
    pltpu.sync_copy(src_hbm.at[pl.ds(base0, _EPW)], sidx)
    pltpu.sync_copy(dst_hbm.at[pl.ds(base0, _EPW)], didx)
    lane = jnp.arange(_LANES, dtype=jnp.int32)

    def start(c, sb, db, ss, sd):
        off = c * _B
        pltpu.async_copy(mu_hbm.at[sidx.at[pl.ds(off, _B)]], sb, ss)
        pltpu.async_copy(mu_hbm.at[didx.at[pl.ds(off, _B)]], db, sd)

    def wait(sb, db, ss, sd):
        pltpu.make_async_copy(mu_hbm.at[sidx.at[pl.ds(0, _B)]], sb, ss).wait()
        pltpu.make_async_copy(mu_hbm.at[didx.at[pl.ds(0, _B)]], db, sd).wait()

    def compute(c, srows, drows):
        off = c * _B
        for g in range(_B // _LANES):
            res = jnp.zeros((_LANES,), jnp.float32)
            for l in range(_LANES):
                e = g * _LANES + l
                acc = None
                for j in range(_D // _LANES):
                    s = srows[e, pl.ds(j * _LANES, _LANES)]
                    t = drows[e, pl.ds(j * _LANES, _LANES)]
                    df = s - t
                    acc = df * df if acc is None else acc + df * df
                for sh in (8, 4, 2, 1):
                    acc = acc + _lane_perm(acc, lane ^ sh)
                res = jnp.where(lane == l, acc, res)
            dist_v[pl.ds(off + g * _LANES, _LANES)] = res

    b0 = (srows0, drows0, sem_s0, sem_d0)
    b1 = (srows1, drows1, sem_s1, sem_d1)
    start(0, *b0)
    start(1, *b1)

    def pair_body(g, carry):
        c0 = 2 * g
        wait(*b0)
        compute(c0, b0[0], b0[1])
        start(c0 + 2, *b0)
        c1 = c0 + 1
        wait(*b1)
        compute(c1, b1[0], b1[1])
        start(jnp.minimum(c1 + 2, _NCHUNK - 1), *b1)
        return carry

    lax.fori_loop(0, (_NCHUNK - 1) // 2, pair_body, 0)
    wait(*b0)
    compute(_NCHUNK - 1, b0[0], b0[1])
    wait(*b1)
    pltpu.sync_copy(dist_v, out_hbm.at[pl.ds(base0, _EPW)])


_GDN = lax.GatherDimensionNumbers(
    offset_dims=(), collapsed_slice_dims=(0,), start_index_map=(0,))


def _lane_perm(v, idx):
    return lax.gather(v, idx[:, None], dimension_numbers=_GDN,
                      slice_sizes=(1,), mode=lax.GatherScatterMode.PROMISE_IN_BOUNDS)


_sc_dist = functools.partial(
    pl.kernel,
    out_type=jax.ShapeDtypeStruct((_E,), jnp.float32),
    mesh=plsc.VectorSubcoreMesh(core_axis_name="c", subcore_axis_name="s"),
    compiler_params=pltpu.CompilerParams(needs_layout_passes=False),
    scratch_types=[
        pltpu.VMEM((_EPW,), jnp.int32),
        pltpu.VMEM((_EPW,), jnp.int32),
        pltpu.VMEM((_B, _D), jnp.float32),
        pltpu.VMEM((_B, _D), jnp.float32),
        pltpu.VMEM((_B, _D), jnp.float32),
        pltpu.VMEM((_B, _D), jnp.float32),
        pltpu.VMEM((_EPW,), jnp.float32),
        pltpu.SemaphoreType.DMA,
        pltpu.SemaphoreType.DMA,
        pltpu.SemaphoreType.DMA,
        pltpu.SemaphoreType.DMA,
    ],
)(_sc_dist_body)


def _tc_epilogue(dist_ref, mu_ref, ls_ref, out_ref):
    d = dist_ref[...]
    p = jax.nn.sigmoid(-d)
    nll = -jnp.sum(jnp.log(p))
    m = mu_ref[...]
    ls = ls_ref[...]
    sig = jnp.exp(ls)
    kl = 0.5 * jnp.sum(sig * sig + m * m - ls - 1.0)
    out_ref[0, 0] = nll + kl


def kernel(edge_index, mu, log_sigma):
    src = edge_index[0]
    dst = edge_index[1]
    dist = _sc_dist(src, dst, mu)
    dist2 = dist.reshape(_E // _D, _D)
    out = pl.pallas_call(
        _tc_epilogue,
        out_specs=pl.BlockSpec(memory_space=pltpu.SMEM),
        out_shape=jax.ShapeDtypeStruct((1, 1), jnp.float32),
    )(dist2, mu, log_sigma)
    return out[0, 0]


# speedup vs baseline: 5.1222x; 1.2140x over previous
import functools

import jax
import jax.numpy as jnp
from jax import lax
from jax.experimental import pallas as pl
from jax.experimental.pallas import tpu as pltpu
from jax.experimental.pallas import tpu_sc as plsc

_N = 10000
_D = 128
_E = 320000
_NC = 2
_NS = 16
_NW = _NC * _NS
_EPW = _E // _NW
_B = 80
_NCHUNK = _EPW // _B
_LANES = 16


def _sc_dist_body(src_hbm, dst_hbm, mu_hbm, out_hbm,
                  sidx, didx, srows0, drows0, srows1, drows1, dist_v,
                  sem_s0, sem_d0, sem_s1, sem_d1):
    wid = lax.axis_index("s") * _NC + lax.axis_index("c")
    base0 = wid * _EPW
    pltpu.sync_copy(src_hbm.at[pl.ds(base0, _EPW)], sidx)
    pltpu.sync_copy(dst_hbm.at[pl.ds(base0, _EPW)], didx)
    lane = jnp.arange(_LANES, dtype=jnp.int32)

    def start(c, sb, db, ss, sd):
        off = c * _B
        pltpu.async_copy(mu_hbm.at[sidx.at[pl.ds(off, _B)]], sb, ss)
        pltpu.async_copy(mu_hbm.at[didx.at[pl.ds(off, _B)]], db, sd)

    def wait(sb, db, ss, sd):
        pltpu.make_async_copy(mu_hbm.at[sidx.at[pl.ds(0, _B)]], sb, ss).wait()
        pltpu.make_async_copy(mu_hbm.at[didx.at[pl.ds(0, _B)]], db, sd).wait()

    def compute(c, srows, drows):
        off = c * _B
        for g in range(_B // _LANES):
            res = jnp.zeros((_LANES,), jnp.float32)
            for l in range(_LANES):
                e = g * _LANES + l
                acc = None
                for j in range(_D // (2 * _LANES)):
                    s2 = plsc.bitcast(srows[e, pl.ds(j * _LANES, _LANES)], jnp.bfloat16)
                    t2 = plsc.bitcast(drows[e, pl.ds(j * _LANES, _LANES)], jnp.bfloat16)
                    s_lo, s_hi = plsc.unpack(s2, format=plsc.PackFormat.INTERLEAVED,
                                             preferred_element_type=jnp.float32)
                    t_lo, t_hi = plsc.unpack(t2, format=plsc.PackFormat.INTERLEAVED,
                                             preferred_element_type=jnp.float32)
                    df0 = s_lo - t_lo
                    df1 = s_hi - t_hi
                    q = df0 * df0 + df1 * df1
                    acc = q if acc is None else acc + q
                for sh in (8, 4, 2, 1):
                    acc = acc + _lane_perm(acc, lane ^ sh)
                res = jnp.where(lane == l, acc, res)
            dist_v[pl.ds(off + g * _LANES, _LANES)] = res

    b0 = (srows0, drows0, sem_s0, sem_d0)
    b1 = (srows1, drows1, sem_s1, sem_d1)
    start(0, *b0)
    start(1, *b1)

    def pair_body(g, carry):
        c0 = 2 * g
        wait(*b0)
        compute(c0, b0[0], b0[1])
        start(c0 + 2, *b0)
        c1 = c0 + 1
        wait(*b1)
        compute(c1, b1[0], b1[1])
        start(jnp.minimum(c1 + 2, _NCHUNK - 1), *b1)
        return carry

    lax.fori_loop(0, (_NCHUNK - 1) // 2, pair_body, 0)
    wait(*b0)
    compute(_NCHUNK - 1, b0[0], b0[1])
    wait(*b1)
    pltpu.sync_copy(dist_v, out_hbm.at[pl.ds(base0, _EPW)])


_GDN = lax.GatherDimensionNumbers(
    offset_dims=(), collapsed_slice_dims=(0,), start_index_map=(0,))


def _lane_perm(v, idx):
    return lax.gather(v, idx[:, None], dimension_numbers=_GDN,
                      slice_sizes=(1,), mode=lax.GatherScatterMode.PROMISE_IN_BOUNDS)


_sc_dist = functools.partial(
    pl.kernel,
    out_type=jax.ShapeDtypeStruct((_E,), jnp.float32),
    mesh=plsc.VectorSubcoreMesh(core_axis_name="c", subcore_axis_name="s"),
    compiler_params=pltpu.CompilerParams(needs_layout_passes=False,
                                         use_tc_tiling_on_sc=False),
    scratch_types=[
        pltpu.VMEM((_EPW,), jnp.int32),
        pltpu.VMEM((_EPW,), jnp.int32),
        pltpu.VMEM((_B, _D // 2), jnp.int32),
        pltpu.VMEM((_B, _D // 2), jnp.int32),
        pltpu.VMEM((_B, _D // 2), jnp.int32),
        pltpu.VMEM((_B, _D // 2), jnp.int32),
        pltpu.VMEM((_EPW,), jnp.float32),
        pltpu.SemaphoreType.DMA,
        pltpu.SemaphoreType.DMA,
        pltpu.SemaphoreType.DMA,
        pltpu.SemaphoreType.DMA,
    ],
)(_sc_dist_body)


def _tc_epilogue(dist_ref, mu_ref, ls_ref, out_ref):
    d = dist_ref[...]
    p = jax.nn.sigmoid(-d)
    nll = -jnp.sum(jnp.log(p))
    m = mu_ref[...]
    ls = ls_ref[...]
    sig = jnp.exp(ls)
    kl = 0.5 * jnp.sum(sig * sig + m * m - ls - 1.0)
    out_ref[0, 0] = nll + kl


def kernel(edge_index, mu, log_sigma):
    src = edge_index[0]
    dst = edge_index[1]
    mu_packed = lax.bitcast_convert_type(
        mu.astype(jnp.bfloat16).reshape(_N, _D // 2, 2), jnp.int32)
    dist = _sc_dist(src, dst, mu_packed)
    dist2 = dist.reshape(_E // _D, _D)
    out = pl.pallas_call(
        _tc_epilogue,
        out_specs=pl.BlockSpec(memory_space=pltpu.SMEM),
        out_shape=jax.ShapeDtypeStruct((1, 1), jnp.float32),
    )(dist2, mu, log_sigma)
    return out[0, 0]
